# baseline (speedup 1.0000x reference)
import functools

import jax
import jax.numpy as jnp
from jax import lax
from jax.experimental import pallas as pl
from jax.experimental.pallas import tpu as pltpu
from jax.experimental.pallas import tpu_sc as plsc

NC = 2
NS = 16
LANES = 128
SLAB = 128
EB = 128


def _mesh():
    return plsc.VectorSubcoreMesh(core_axis_name="c", subcore_axis_name="s")


def _make_deg(n_pad, e_pad):
    per_tile = e_pad // (NC * NS)
    nb = per_tile // EB
    rows_per_tile = n_pad // NS

    @functools.partial(
        pl.kernel,
        mesh=_mesh(),
        out_type=[jax.ShapeDtypeStruct((n_pad, LANES), jnp.float32)] * NC,
        scratch_types=[
            pltpu.VMEM((EB,), jnp.int32),
            pltpu.VMEM((EB,), jnp.int32),
            pltpu.VMEM((EB, LANES), jnp.float32),
            pltpu.VMEM_SHARED((n_pad, LANES), jnp.float32),
            pltpu.SemaphoreType.DMA,
            pltpu.SemaphoreType.DMA,
        ],
    )
    def deg_kernel(dst_hbm, ones_hbm, zeros_hbm, out0, out1,
                   didx0, didx1, ones_v, spm, sem0, sem1):
        c = lax.axis_index("c")
        s = lax.axis_index("s")
        wid = c * NS + s
        r0 = s * rows_per_tile
        base = wid * per_tile
        pltpu.sync_copy(zeros_hbm.at[pl.ds(0, rows_per_tile)],
                        spm.at[pl.ds(r0, rows_per_tile)])
        pltpu.sync_copy(ones_hbm, ones_v)
        plsc.subcore_barrier()

        pltpu.async_copy(dst_hbm.at[pl.ds(base, EB)], didx0, sem0)

        def body(b2, carry):
            b0 = b2 * 2
            b1 = b0 + 1
            pltpu.async_copy(dst_hbm.at[pl.ds(base + b1 * EB, EB)],
                             didx1, sem1)
            pltpu.make_async_copy(dst_hbm.at[pl.ds(base + b0 * EB, EB)],
                                  didx0, sem0).wait()
            pltpu.sync_copy(ones_v, spm.at[didx0], add=True)

            @pl.when(b2 + 1 < nb // 2)
            def _():
                pltpu.async_copy(dst_hbm.at[pl.ds(base + (b0 + 2) * EB, EB)],
                                 didx0, sem0)

            pltpu.make_async_copy(dst_hbm.at[pl.ds(base + b1 * EB, EB)],
                                  didx1, sem1).wait()
            pltpu.sync_copy(ones_v, spm.at[didx1], add=True)
            return carry

        lax.fori_loop(0, nb // 2, body, 0)
        plsc.subcore_barrier()

        @pl.when(c == 0)
        def _():
            pltpu.sync_copy(spm.at[pl.ds(r0, rows_per_tile)],
                            out0.at[pl.ds(r0, rows_per_tile)])

        @pl.when(c == 1)
        def _():
            pltpu.sync_copy(spm.at[pl.ds(r0, rows_per_tile)],
                            out1.at[pl.ds(r0, rows_per_tile)])

    return deg_kernel


def _make_scat(n_pad, e_pad, n_slabs):
    per_tile = e_pad // NS
    nb = per_tile // EB
    G = 16
    ngroups = nb // G
    rows_per_tile = n_pad // NS

    @functools.partial(
        pl.kernel,
        mesh=_mesh(),
        out_type=[jax.ShapeDtypeStruct((n_pad, SLAB), jnp.float32)] * n_slabs,
        scratch_types=[
            pltpu.VMEM((4 * G, EB // 4), jnp.int32),
            pltpu.VMEM((G, EB), jnp.int32),
            pltpu.VMEM((EB, SLAB), jnp.float32),
            pltpu.VMEM((EB, SLAB), jnp.float32),
            pltpu.VMEM_SHARED((n_pad, SLAB), jnp.float32),
            pltpu.SemaphoreType.DMA,
            pltpu.SemaphoreType.DMA,
        ],
    )
    def scat_kernel(src_hbm, dst_hbm, *refs):
        g_refs = refs[:n_slabs]
        out_refs = refs[n_slabs:2 * n_slabs]
        sidx, didx, rows0, rows1, spm, semA, semB = refs[2 * n_slabs:]
        c = lax.axis_index("c")
        s = lax.axis_index("s")
        r0 = s * rows_per_tile

        def do_slab(g_hbm, out_hbm):
            pltpu.sync_copy(g_hbm.at[pl.ds(r0, rows_per_tile)],
                            spm.at[pl.ds(r0, rows_per_tile)])
            plsc.subcore_barrier()

            def fire(b, buf, sem):
                h = EB // 4
                for q in range(4):
                    pltpu.async_copy(g_hbm.at[sidx.at[4 * b + q]],
                                     buf.at[pl.ds(q * h, h)], sem)

            def drain(b, buf, sem):
                h = EB // 4
                for q in range(4):
                    pltpu.make_async_copy(g_hbm.at[sidx.at[4 * b + q]],
                                          buf.at[pl.ds(q * h, h)],
                                          sem).wait()

            def group(gi, carry):
                pltpu.sync_copy(src_hbm.at[pl.ds(4 * (s * nb + gi * G),
                                                 4 * G)], sidx)
                pltpu.sync_copy(dst_hbm.at[pl.ds(s * nb + gi * G, G)], didx)
                fire(0, rows0, semA)

                def body(b2, carry2):
                    b0 = b2 * 2
                    b1 = b0 + 1
                    fire(b1, rows1, semB)
                    drain(b0, rows0, semA)
                    pltpu.sync_copy(rows0, spm.at[didx.at[b0]], add=True)

                    @pl.when(b2 + 1 < G // 2)
                    def _():
                        fire(b0 + 2, rows0, semA)

                    drain(b1, rows1, semB)
                    pltpu.sync_copy(rows1, spm.at[didx.at[b1]], add=True)
                    return carry2

                lax.fori_loop(0, G // 2, body, 0)
                return carry

            lax.fori_loop(0, ngroups, group, 0)
            plsc.subcore_barrier()
            pltpu.sync_copy(spm.at[pl.ds(r0, rows_per_tile)],
                            out_hbm.at[pl.ds(r0, rows_per_tile)])

        for slab in range(n_slabs):
            pl.when(c == (slab % NC))(
                functools.partial(do_slab, g_refs[slab], out_refs[slab]))

    return scat_kernel


def _slab_spec(rb):
    return pl.BlockSpec((rb, LANES), lambda i: (i, 0))


def _pre_kernel(degA, degB, *refs):
    nsl = (len(refs) - 1) // 2
    x_refs = refs[:nsl]
    dinvb = refs[nsl]
    g_refs = refs[nsl + 1:]
    d = degA[...] + degB[...] + 1.0
    dv = lax.rsqrt(d)
    dinvb[...] = dv
    for k in range(nsl):
        g_refs[k][...] = x_refs[k][...] * dv[:, 0:1]


def _run_pre(degA, degB, x_slabs, n_pad, rb):
    nsl = len(x_slabs)
    grid = n_pad // rb
    out = pl.pallas_call(
        _pre_kernel,
        grid=(grid,),
        in_specs=[_slab_spec(rb)] * (2 + nsl),
        out_specs=[_slab_spec(rb)] * (1 + nsl),
        out_shape=[jax.ShapeDtypeStruct((n_pad, LANES), jnp.float32)]
        * (1 + nsl),
    )(degA, degB, *x_slabs)
    return out[0], out[1:]


def _mm_kernel(fused, n_in, n_out, *refs):
    lhs = refs[:n_in]
    i = n_in
    if fused:
        dv = refs[i][:, 0:1]
        i += 1
    w_ref = refs[i]
    b_ref = refs[i + 1]
    outs = refs[i + 2:]
    a = jnp.concatenate([r[...] for r in lhs], axis=1)
    if fused:
        a = a * dv
    y = jnp.dot(a, w_ref[...], preferred_element_type=jnp.float32)
    y = y + b_ref[0:1, :]
    if fused:
        y = jnp.maximum(y, 0.0) * dv
    for k in range(n_out):
        outs[k][...] = y[:, k * LANES:(k + 1) * LANES]


def _run_mm(s_slabs, dinvb, w, bias, n_pad, rb, fused):
    cin, cout = w.shape
    n_in, n_out = cin // LANES, cout // LANES
    grid = n_pad // rb
    biasb = jnp.broadcast_to(bias[None, :], (8, cout))
    args = list(s_slabs)
    specs = [_slab_spec(rb)] * n_in
    if fused:
        args.append(dinvb)
        specs.append(_slab_spec(rb))
    args += [w, biasb]
    specs += [pl.BlockSpec((cin, cout), lambda i: (0, 0)),
              pl.BlockSpec((8, cout), lambda i: (0, 0))]
    return pl.pallas_call(
        functools.partial(_mm_kernel, fused, n_in, n_out),
        grid=(grid,),
        in_specs=specs,
        out_specs=[_slab_spec(rb)] * n_out,
        out_shape=[jax.ShapeDtypeStruct((n_pad, LANES), jnp.float32)] * n_out,
    )(*args)


def _post_kernel(n_in, *refs):
    s_refs = refs[:n_in]
    dinv_ref = refs[n_in]
    b_ref = refs[n_in + 1]
    o_ref = refs[n_in + 2]
    a = jnp.concatenate([r[...] for r in s_refs], axis=1)
    o_ref[...] = a * dinv_ref[:, 0:1] + b_ref[0:1, :]


def _run_post(s_slabs, dinvb, bias, n, rb):
    n_in = len(s_slabs)
    c = n_in * LANES
    grid = n // rb
    biasb = jnp.broadcast_to(bias[None, :], (8, c))
    return pl.pallas_call(
        functools.partial(_post_kernel, n_in),
        grid=(grid,),
        in_specs=[_slab_spec(rb)] * (n_in + 1)
        + [pl.BlockSpec((8, c), lambda i: (0, 0))],
        out_specs=pl.BlockSpec((rb, c), lambda i: (i, 0)),
        out_shape=jax.ShapeDtypeStruct((n, c), jnp.float32),
    )(*s_slabs, dinvb, biasb)


def kernel(x, edge_index, W1, b1, W2, b2, W3, b3):
    n = x.shape[0]
    e = edge_index.shape[1]
    n_pad = ((n + NS - 1) // NS + 7) // 8 * 8 * NS
    batch_stride = NC * NS * EB
    e_pad = ((e + batch_stride - 1) // batch_stride) * batch_stride
    rbp = n_pad // 4
    rb = 2000 if n % 2000 == 0 else (1000 if n % 1000 == 0 else 8)

    src = edge_index[0].astype(jnp.int32)
    dst = edge_index[1].astype(jnp.int32)
    pad_e = e_pad - e
    src_p = jnp.concatenate([src, jnp.zeros((pad_e,), jnp.int32)])
    dst_p = jnp.concatenate([dst, jnp.full((pad_e,), n, jnp.int32)])
    src2 = src_p.reshape(-1, EB // 4)
    dst2 = dst_p.reshape(-1, EB)

    def scat(slabs):
        return list(_make_scat(n_pad, e_pad, len(slabs))(src2, dst2, *slabs))

    ones_hbm = jnp.ones((EB, LANES), jnp.float32)
    zeros_hbm = jnp.zeros((n_pad // NS, LANES), jnp.float32)
    degA, degB = _make_deg(n_pad, e_pad)(dst_p, ones_hbm, zeros_hbm)

    x_pad = jnp.concatenate(
        [x, jnp.zeros((n_pad - n, x.shape[1]), jnp.float32)], axis=0)
    x_slabs = [x_pad[:, k * LANES:(k + 1) * LANES]
               for k in range(x.shape[1] // LANES)]
    dinvb, g0 = _run_pre(degA, degB, x_slabs, n_pad, rbp)

    s1 = scat(g0)
    g1 = _run_mm(s1, dinvb, W1, b1, n_pad, rbp, fused=True)
    s2 = scat(g1)
    g2 = _run_mm(s2, dinvb, W2, b2, n_pad, rbp, fused=True)
    u = _run_mm(g2, None, W3, jnp.zeros((W3.shape[1],), jnp.float32),
                n_pad, rbp, fused=False)
    s3 = scat(u)
    return _run_post(s3, dinvb, b3, n, rb)


# speedup vs baseline: 1.1650x; 1.1650x over previous
import functools

import jax
import jax.numpy as jnp
from jax import lax
from jax.experimental import pallas as pl
from jax.experimental.pallas import tpu as pltpu
from jax.experimental.pallas import tpu_sc as plsc

NC = 2
NS = 16
LANES = 128
SLAB = 128
EB = 128


def _mesh():
    return plsc.VectorSubcoreMesh(core_axis_name="c", subcore_axis_name="s")


def _make_deg(n_pad, e_pad):
    per_tile = e_pad // (NC * NS)
    nb = per_tile // EB
    rows_per_tile = n_pad // NS

    @functools.partial(
        pl.kernel,
        mesh=_mesh(),
        out_type=[jax.ShapeDtypeStruct((n_pad, LANES), jnp.float32)] * NC,
        scratch_types=[
            pltpu.VMEM((EB,), jnp.int32),
            pltpu.VMEM((EB,), jnp.int32),
            pltpu.VMEM((EB, LANES), jnp.float32),
            pltpu.VMEM_SHARED((n_pad, LANES), jnp.float32),
            pltpu.SemaphoreType.DMA,
            pltpu.SemaphoreType.DMA,
        ],
    )
    def deg_kernel(dst_hbm, ones_hbm, zeros_hbm, out0, out1,
                   didx0, didx1, ones_v, spm, sem0, sem1):
        c = lax.axis_index("c")
        s = lax.axis_index("s")
        wid = c * NS + s
        r0 = s * rows_per_tile
        base = wid * per_tile
        pltpu.sync_copy(zeros_hbm.at[pl.ds(0, rows_per_tile)],
                        spm.at[pl.ds(r0, rows_per_tile)])
        pltpu.sync_copy(ones_hbm, ones_v)
        plsc.subcore_barrier()

        pltpu.async_copy(dst_hbm.at[pl.ds(base, EB)], didx0, sem0)

        def body(b2, carry):
            b0 = b2 * 2
            b1 = b0 + 1
            pltpu.async_copy(dst_hbm.at[pl.ds(base + b1 * EB, EB)],
                             didx1, sem1)
            pltpu.make_async_copy(dst_hbm.at[pl.ds(base + b0 * EB, EB)],
                                  didx0, sem0).wait()
            pltpu.sync_copy(ones_v, spm.at[didx0], add=True)

            @pl.when(b2 + 1 < nb // 2)
            def _():
                pltpu.async_copy(dst_hbm.at[pl.ds(base + (b0 + 2) * EB, EB)],
                                 didx0, sem0)

            pltpu.make_async_copy(dst_hbm.at[pl.ds(base + b1 * EB, EB)],
                                  didx1, sem1).wait()
            pltpu.sync_copy(ones_v, spm.at[didx1], add=True)
            return carry

        lax.fori_loop(0, nb // 2, body, 0)
        plsc.subcore_barrier()

        @pl.when(c == 0)
        def _():
            pltpu.sync_copy(spm.at[pl.ds(r0, rows_per_tile)],
                            out0.at[pl.ds(r0, rows_per_tile)])

        @pl.when(c == 1)
        def _():
            pltpu.sync_copy(spm.at[pl.ds(r0, rows_per_tile)],
                            out1.at[pl.ds(r0, rows_per_tile)])

    return deg_kernel


def _make_scat(n_pad, e_pad, n_slabs):
    per_tile = e_pad // NS
    nb = per_tile // EB
    G = 16
    ngroups = nb // G
    rows_per_tile = n_pad // NS

    @functools.partial(
        pl.kernel,
        mesh=_mesh(),
        out_type=[jax.ShapeDtypeStruct((n_pad, SLAB), jnp.float32)] * n_slabs,
        scratch_types=[
            [pltpu.VMEM((2 * G, EB // 2), jnp.int32)] * 2,
            [pltpu.VMEM((G, EB), jnp.int32)] * 2,
            pltpu.SemaphoreType.DMA,
            pltpu.VMEM((EB, SLAB), jnp.float32),
            pltpu.VMEM((EB, SLAB), jnp.float32),
            pltpu.VMEM_SHARED((n_pad, SLAB), jnp.float32),
            pltpu.SemaphoreType.DMA,
            pltpu.SemaphoreType.DMA,
        ],
    )
    def scat_kernel(src_hbm, dst_hbm, *refs):
        g_refs = refs[:n_slabs]
        out_refs = refs[n_slabs:2 * n_slabs]
        sidxs, didxs, semI, rows0, rows1, spm, semA, semB = refs[2 * n_slabs:]
        c = lax.axis_index("c")
        s = lax.axis_index("s")
        r0 = s * rows_per_tile

        def do_slab(g_hbm, out_hbm):
            pltpu.sync_copy(g_hbm.at[pl.ds(r0, rows_per_tile)],
                            spm.at[pl.ds(r0, rows_per_tile)])
            plsc.subcore_barrier()

            def fire(b, buf, sem, sidx):
                h = EB // 2
                pltpu.async_copy(g_hbm.at[sidx.at[2 * b]],
                                 buf.at[pl.ds(0, h)], sem)
                pltpu.async_copy(g_hbm.at[sidx.at[2 * b + 1]],
                                 buf.at[pl.ds(h, h)], sem)

            def drain(b, buf, sem, sidx):
                h = EB // 2
                pltpu.make_async_copy(g_hbm.at[sidx.at[2 * b]],
                                      buf.at[pl.ds(0, h)], sem).wait()
                pltpu.make_async_copy(g_hbm.at[sidx.at[2 * b + 1]],
                                      buf.at[pl.ds(h, h)], sem).wait()

            def stage(gi, p):
                pltpu.async_copy(src_hbm.at[pl.ds(2 * (s * nb + gi * G),
                                                  2 * G)], sidxs[p], semI)
                pltpu.async_copy(dst_hbm.at[pl.ds(s * nb + gi * G, G)],
                                 didxs[p], semI)

            def stage_wait(gi, p):
                pltpu.make_async_copy(src_hbm.at[pl.ds(2 * (s * nb + gi * G),
                                                       2 * G)], sidxs[p],
                                      semI).wait()
                pltpu.make_async_copy(dst_hbm.at[pl.ds(s * nb + gi * G, G)],
                                      didxs[p], semI).wait()

            stage(0, 0)
            for gi in range(ngroups):
                p = gi % 2
                sidx, didx = sidxs[p], didxs[p]
                stage_wait(gi, p)
                if gi + 1 < ngroups:
                    stage(gi + 1, 1 - p)
                fire(0, rows0, semA, sidx)

                def body(b2, carry2, sidx=sidx, didx=didx):
                    b0 = b2 * 2
                    b1 = b0 + 1
                    fire(b1, rows1, semB, sidx)
                    drain(b0, rows0, semA, sidx)
                    pltpu.sync_copy(rows0, spm.at[didx.at[b0]], add=True)

                    @pl.when(b2 + 1 < G // 2)
                    def _():
                        fire(b0 + 2, rows0, semA, sidx)

                    drain(b1, rows1, semB, sidx)
                    pltpu.sync_copy(rows1, spm.at[didx.at[b1]], add=True)
                    return carry2

                lax.fori_loop(0, G // 2, body, 0)
            plsc.subcore_barrier()
            pltpu.sync_copy(spm.at[pl.ds(r0, rows_per_tile)],
                            out_hbm.at[pl.ds(r0, rows_per_tile)])

        for slab in range(n_slabs):
            pl.when(c == (slab % NC))(
                functools.partial(do_slab, g_refs[slab], out_refs[slab]))

    return scat_kernel


def _slab_spec(rb):
    return pl.BlockSpec((rb, LANES), lambda i: (i, 0))


def _pre_kernel(degA, degB, *refs):
    nsl = (len(refs) - 1) // 2
    x_refs = refs[:nsl]
    dinvb = refs[nsl]
    g_refs = refs[nsl + 1:]
    d = degA[...] + degB[...] + 1.0
    dv = lax.rsqrt(d)
    dinvb[...] = dv
    for k in range(nsl):
        g_refs[k][...] = x_refs[k][...] * dv[:, 0:1]


def _run_pre(degA, degB, x_slabs, n_pad, rb):
    nsl = len(x_slabs)
    grid = n_pad // rb
    out = pl.pallas_call(
        _pre_kernel,
        grid=(grid,),
        in_specs=[_slab_spec(rb)] * (2 + nsl),
        out_specs=[_slab_spec(rb)] * (1 + nsl),
        out_shape=[jax.ShapeDtypeStruct((n_pad, LANES), jnp.float32)]
        * (1 + nsl),
    )(degA, degB, *x_slabs)
    return out[0], out[1:]


def _mm_kernel(fused, n_in, n_out, *refs):
    lhs = refs[:n_in]
    i = n_in
    if fused:
        dv = refs[i][:, 0:1]
        i += 1
    w_ref = refs[i]
    b_ref = refs[i + 1]
    outs = refs[i + 2:]
    a = jnp.concatenate([r[...] for r in lhs], axis=1)
    if fused:
        a = a * dv
    y = jnp.dot(a, w_ref[...], preferred_element_type=jnp.float32)
    y = y + b_ref[0:1, :]
    if fused:
        y = jnp.maximum(y, 0.0) * dv
    for k in range(n_out):
        outs[k][...] = y[:, k * LANES:(k + 1) * LANES]


def _run_mm(s_slabs, dinvb, w, bias, n_pad, rb, fused):
    cin, cout = w.shape
    n_in, n_out = cin // LANES, cout // LANES
    grid = n_pad // rb
    biasb = jnp.broadcast_to(bias[None, :], (8, cout))
    args = list(s_slabs)
    specs = [_slab_spec(rb)] * n_in
    if fused:
        args.append(dinvb)
        specs.append(_slab_spec(rb))
    args += [w, biasb]
    specs += [pl.BlockSpec((cin, cout), lambda i: (0, 0)),
              pl.BlockSpec((8, cout), lambda i: (0, 0))]
    return pl.pallas_call(
        functools.partial(_mm_kernel, fused, n_in, n_out),
        grid=(grid,),
        in_specs=specs,
        out_specs=[_slab_spec(rb)] * n_out,
        out_shape=[jax.ShapeDtypeStruct((n_pad, LANES), jnp.float32)] * n_out,
    )(*args)


def _post_kernel(n_in, *refs):
    s_refs = refs[:n_in]
    dinv_ref = refs[n_in]
    b_ref = refs[n_in + 1]
    o_ref = refs[n_in + 2]
    a = jnp.concatenate([r[...] for r in s_refs], axis=1)
    o_ref[...] = a * dinv_ref[:, 0:1] + b_ref[0:1, :]


def _run_post(s_slabs, dinvb, bias, n, rb):
    n_in = len(s_slabs)
    c = n_in * LANES
    grid = n // rb
    biasb = jnp.broadcast_to(bias[None, :], (8, c))
    return pl.pallas_call(
        functools.partial(_post_kernel, n_in),
        grid=(grid,),
        in_specs=[_slab_spec(rb)] * (n_in + 1)
        + [pl.BlockSpec((8, c), lambda i: (0, 0))],
        out_specs=pl.BlockSpec((rb, c), lambda i: (i, 0)),
        out_shape=jax.ShapeDtypeStruct((n, c), jnp.float32),
    )(*s_slabs, dinvb, biasb)


def kernel(x, edge_index, W1, b1, W2, b2, W3, b3):
    n = x.shape[0]
    e = edge_index.shape[1]
    n_pad = ((n + NS - 1) // NS + 7) // 8 * 8 * NS
    batch_stride = NC * NS * EB
    e_pad = ((e + batch_stride - 1) // batch_stride) * batch_stride
    rbp = n_pad // 4
    rb = 2000 if n % 2000 == 0 else (1000 if n % 1000 == 0 else 8)

    src = edge_index[0].astype(jnp.int32)
    dst = edge_index[1].astype(jnp.int32)
    pad_e = e_pad - e
    src_p = jnp.concatenate([src, jnp.zeros((pad_e,), jnp.int32)])
    dst_p = jnp.concatenate([dst, jnp.full((pad_e,), n, jnp.int32)])
    src2 = src_p.reshape(-1, EB // 2)
    dst2 = dst_p.reshape(-1, EB)

    def scat(slabs):
        return list(_make_scat(n_pad, e_pad, len(slabs))(src2, dst2, *slabs))

    ones_hbm = jnp.ones((EB, LANES), jnp.float32)
    zeros_hbm = jnp.zeros((n_pad // NS, LANES), jnp.float32)
    degA, degB = _make_deg(n_pad, e_pad)(dst_p, ones_hbm, zeros_hbm)

    x_pad = jnp.concatenate(
        [x, jnp.zeros((n_pad - n, x.shape[1]), jnp.float32)], axis=0)
    x_slabs = [x_pad[:, k * LANES:(k + 1) * LANES]
               for k in range(x.shape[1] // LANES)]
    dinvb, g0 = _run_pre(degA, degB, x_slabs, n_pad, rbp)

    s1 = scat(g0)
    g1 = _run_mm(s1, dinvb, W1, b1, n_pad, rbp, fused=True)
    s2 = scat(g1)
    g2 = _run_mm(s2, dinvb, W2, b2, n_pad, rbp, fused=True)
    u = _run_mm(g2, None, W3, jnp.zeros((W3.shape[1],), jnp.float32),
                n_pad, rbp, fused=False)
    s3 = scat(u)
    return _run_post(s3, dinvb, b3, n, rb)
